# Initial kernel scaffold; baseline (speedup 1.0000x reference)
import jax
import jax.numpy as jnp
from jax.experimental import pallas as pl


def kernel(scientist_ids, paper_ids, scientist_factors, paper_factors, scientist_bias, paper_bias, implicit_factors, implicit_wishlist, global_bias, s2p_flat, s2p_cu, s2w_flat, s2w_cu):
    raise NotImplementedError("write your pallas kernel here")


# speedup vs baseline: 72.0653x; 72.0653x over previous
import functools

import numpy as np
import jax
import jax.numpy as jnp
from jax import lax
from jax.experimental import pallas as pl
from jax.experimental.pallas import tpu as pltpu
from jax.experimental.pallas import tpu_sc as plsc

B = 4096
EMB = 64
NC, NS = 2, 16
NW = NC * NS
UPW = B // NW
MAXP, MAXW = 100, 40
P_ROWS = 112
W_ROWS = 48
IDXP = UPW * MAXP + 8 + P_ROWS
IDXW = UPW * MAXW + 8 + W_ROWS
CU_COPY = 136
CU_LEN = 144
LUT_LEN = 120


def _sread(ref, i):
    return ref[pl.ds(i, 16)][0]


def _body(sids, pids, sfac, pfac, sbias, pbias, ifac, iwl, gb, s2p, s2pcu,
          s2w, s2wcu, lut, out_hbm,
          idxp_v, idxw_v, cup_v, cuw_v, sid_v, pid_v, bsid_v, bpid_v,
          u_v, p_v, bu_v, bp_v,
          lut_v, gb_v, out_v, rows_p, rows_w, sem_a, sem_b, sem_pro):
    cid = lax.axis_index("c")
    sid = lax.axis_index("s")
    wid = sid * NC + cid
    base = pl.multiple_of(wid * UPW, UPW)

    pltpu.sync_copy(s2pcu.at[pl.ds(base, CU_COPY)], cup_v.at[pl.ds(0, CU_COPY)])
    pltpu.sync_copy(s2wcu.at[pl.ds(base, CU_COPY)], cuw_v.at[pl.ds(0, CU_COPY)])
    pltpu.sync_copy(sids.at[pl.ds(base, UPW)], sid_v)
    pltpu.sync_copy(pids.at[pl.ds(base, UPW)], pid_v)
    pltpu.sync_copy(lut, lut_v)
    pltpu.sync_copy(gb, gb_v)

    startp = pl.multiple_of(_sread(cup_v, 0) & jnp.int32(-8), 8)
    startw = pl.multiple_of(_sread(cuw_v, 0) & jnp.int32(-8), 8)
    pltpu.sync_copy(s2p.at[pl.ds(startp, IDXP)], idxp_v)
    pltpu.sync_copy(s2w.at[pl.ds(startw, IDXW)], idxw_v)

    for k in range(UPW // 16):
        sl = pl.ds(k * 16, 16)
        bsid_v[sl] = lax.shift_right_logical(sid_v[sl], 4)
        bpid_v[sl] = lax.shift_right_logical(pid_v[sl], 4)

    cp_u = pltpu.async_copy(sfac.at[sid_v], u_v, sem_pro)
    cp_p = pltpu.async_copy(pfac.at[pid_v], p_v, sem_pro)
    cp_bu = pltpu.async_copy(sbias.at[bsid_v], bu_v, sem_pro)
    cp_bp = pltpu.async_copy(pbias.at[bpid_v], bp_v, sem_pro)
    cp_u.wait()
    cp_p.wait()
    cp_bu.wait()
    cp_bp.wait()

    lane = lax.iota(jnp.int32, 16)
    lane0 = lane == 0

    def issue(u, slot):
        sem = sem_a if slot == 0 else sem_b
        offp = pl.multiple_of((_sread(cup_v, u) - startp) & jnp.int32(-8), 8)
        offw = pl.multiple_of((_sread(cuw_v, u) - startw) & jnp.int32(-8), 8)
        pltpu.async_copy(ifac.at[idxp_v.at[pl.ds(offp, P_ROWS)]],
                         rows_p.at[slot], sem)
        pltpu.async_copy(iwl.at[idxw_v.at[pl.ds(offw, W_ROWS)]],
                         rows_w.at[slot], sem)

    def seg_sum(rows, slot, r0, n):
        zero = jnp.zeros((16,), jnp.float32)

        def bd(j, acc):
            jr = r0 + j
            return (acc[0] + rows[slot, jr, pl.ds(0, 16)],
                    acc[1] + rows[slot, jr, pl.ds(16, 16)],
                    acc[2] + rows[slot, jr, pl.ds(32, 16)],
                    acc[3] + rows[slot, jr, pl.ds(48, 16)])

        return lax.fori_loop(0, n, bd, (zero, zero, zero, zero))

    def consume(u, slot):
        sem = sem_a if slot == 0 else sem_b
        pltpu.make_async_copy(ifac.at[pl.ds(0, P_ROWS)],
                              rows_p.at[slot], sem).wait()
        pltpu.make_async_copy(iwl.at[pl.ds(0, W_ROWS)],
                              rows_w.at[slot], sem).wait()
        sp = _sread(cup_v, u)
        lenp = _sread(cup_v, u + 1) - sp
        r0p = (sp - startp) & jnp.int32(7)
        sw = _sread(cuw_v, u)
        lenw = _sread(cuw_v, u + 1) - sw
        r0w = (sw - startw) & jnp.int32(7)
        accp = seg_sum(rows_p, slot, r0p, lenp)
        accw = seg_sum(rows_w, slot, r0w, lenw)
        rsp = _sread(lut_v, lenp)
        rsw = _sread(lut_v, lenw)
        tacc = jnp.zeros((16,), jnp.float32)
        for ci in range(4):
            sl = pl.ds(ci * 16, 16)
            y = accp[ci] * rsp + accw[ci] * rsw + u_v[u, sl]
            tacc = tacc + y * p_v[u, sl]
        dot = jnp.full((16,), jnp.sum(tacc))
        plsc.store_scatter(out_v, [jnp.full((16,), u, jnp.int32)], dot,
                           mask=lane0)

    issue(jnp.int32(0), 0)

    def outer(g, carry):
        for par in range(2):
            u = g * 2 + par

            @pl.when(u + 1 < UPW)
            def _():
                issue(u + 1, 1 - par)

            consume(u, par)
        return carry

    lax.fori_loop(0, UPW // 2, outer, 0)

    gbias = _sread(gb_v, 0)
    mask15 = jnp.full((16,), 15, jnp.int32)
    for k in range(UPW // 16):
        sl = pl.ds(k * 16, 16)
        rows16 = lane + (k * 16)
        bu16 = plsc.load_gather(bu_v, [rows16, sid_v[sl] & mask15])
        bp16 = plsc.load_gather(bp_v, [rows16, pid_v[sl] & mask15])
        out_v[sl] = out_v[sl] + bu16 + bp16 + gbias

    pltpu.sync_copy(out_v, out_hbm.at[pl.ds(base, UPW)])


_scall = functools.partial(
    pl.kernel,
    out_type=jax.ShapeDtypeStruct((B,), jnp.float32),
    mesh=plsc.VectorSubcoreMesh(core_axis_name="c", subcore_axis_name="s",
                                num_cores=NC, num_subcores=NS),
    compiler_params=pltpu.CompilerParams(needs_layout_passes=False, use_tc_tiling_on_sc=False),
    scratch_types=[
        pltpu.VMEM((IDXP,), jnp.int32),
        pltpu.VMEM((IDXW,), jnp.int32),
        pltpu.VMEM((CU_LEN,), jnp.int32),
        pltpu.VMEM((CU_LEN,), jnp.int32),
        pltpu.VMEM((UPW,), jnp.int32),
        pltpu.VMEM((UPW,), jnp.int32),
        pltpu.VMEM((UPW,), jnp.int32),
        pltpu.VMEM((UPW,), jnp.int32),
        pltpu.VMEM((UPW, EMB), jnp.float32),
        pltpu.VMEM((UPW, EMB), jnp.float32),
        pltpu.VMEM((UPW, 16), jnp.float32),
        pltpu.VMEM((UPW, 16), jnp.float32),
        pltpu.VMEM((LUT_LEN,), jnp.float32),
        pltpu.VMEM((16,), jnp.float32),
        pltpu.VMEM((UPW,), jnp.float32),
        pltpu.VMEM((2, P_ROWS, EMB), jnp.float32),
        pltpu.VMEM((2, W_ROWS, EMB), jnp.float32),
        pltpu.SemaphoreType.DMA,
        pltpu.SemaphoreType.DMA,
        pltpu.SemaphoreType.DMA,
    ],
)(_body)


def kernel(scientist_ids, paper_ids, scientist_factors, paper_factors,
           scientist_bias, paper_bias, implicit_factors, implicit_wishlist,
           global_bias, s2p_flat, s2p_cu, s2w_flat, s2w_cu):
    s2p_pad = jnp.pad(s2p_flat, (0, IDXP))
    s2w_pad = jnp.pad(s2w_flat, (0, IDXW))
    cup_pad = jnp.pad(s2p_cu, (0, CU_LEN))
    cuw_pad = jnp.pad(s2w_cu, (0, CU_LEN))
    gb_pad = jnp.pad(global_bias, (0, 15))
    lut = jnp.asarray(
        1.0 / np.sqrt(np.maximum(np.arange(LUT_LEN), 1)), jnp.float32)
    sb16 = scientist_bias.reshape(-1, 16)
    pb16 = paper_bias.reshape(-1, 16)
    return _scall(scientist_ids.astype(jnp.int32), paper_ids.astype(jnp.int32),
                  scientist_factors, paper_factors, sb16, pb16,
                  implicit_factors, implicit_wishlist, gb_pad, s2p_pad,
                  cup_pad, s2w_pad, cuw_pad, lut)


# speedup vs baseline: 80.2445x; 1.1135x over previous
import functools

import numpy as np
import jax
import jax.numpy as jnp
from jax import lax
from jax.experimental import pallas as pl
from jax.experimental.pallas import tpu as pltpu
from jax.experimental.pallas import tpu_sc as plsc

B = 4096
EMB = 64
NC, NS = 2, 16
NW = NC * NS
UPW = B // NW
MAXP, MAXW = 100, 40
P_ROWS = 112
W_ROWS = 56
NSLOT = 4
IDXP = UPW * MAXP + 8 + P_ROWS
IDXW = UPW * MAXW + 8 + W_ROWS
CU_COPY = 136
CU_LEN = 144
LUT_LEN = 120


def _sread(ref, i):
    return ref[pl.ds(i, 16)][0]


def _body(sids, pids, sfac, pfac, sbias, pbias, ifac, iwl, gb, s2p, s2pcu,
          s2w, s2wcu, lut, out_hbm,
          idxp_v, idxw_v, cup_v, cuw_v, sid_v, pid_v, bsid_v, bpid_v,
          u_v, p_v, bu_v, bp_v,
          lut_v, gb_v, out_v, rows_p, rows_w,
          sem0, sem1, sem2, sem3, sem_pro):
    sems = (sem0, sem1, sem2, sem3)
    cid = lax.axis_index("c")
    sid = lax.axis_index("s")
    wid = sid * NC + cid
    base = pl.multiple_of(wid * UPW, UPW)

    pltpu.sync_copy(s2pcu.at[pl.ds(base, CU_COPY)], cup_v.at[pl.ds(0, CU_COPY)])
    pltpu.sync_copy(s2wcu.at[pl.ds(base, CU_COPY)], cuw_v.at[pl.ds(0, CU_COPY)])
    pltpu.sync_copy(sids.at[pl.ds(base, UPW)], sid_v)
    pltpu.sync_copy(pids.at[pl.ds(base, UPW)], pid_v)
    pltpu.sync_copy(lut, lut_v)
    pltpu.sync_copy(gb, gb_v)

    startp = pl.multiple_of(_sread(cup_v, 0) & jnp.int32(-8), 8)
    startw = pl.multiple_of(_sread(cuw_v, 0) & jnp.int32(-8), 8)
    pltpu.sync_copy(s2p.at[pl.ds(startp, IDXP)], idxp_v)
    pltpu.sync_copy(s2w.at[pl.ds(startw, IDXW)], idxw_v)

    for k in range(UPW // 16):
        sl = pl.ds(k * 16, 16)
        bsid_v[sl] = lax.shift_right_logical(sid_v[sl], 4)
        bpid_v[sl] = lax.shift_right_logical(pid_v[sl], 4)

    cp_u = pltpu.async_copy(sfac.at[sid_v], u_v, sem_pro)
    cp_p = pltpu.async_copy(pfac.at[pid_v], p_v, sem_pro)
    cp_bu = pltpu.async_copy(sbias.at[bsid_v], bu_v, sem_pro)
    cp_bp = pltpu.async_copy(pbias.at[bpid_v], bp_v, sem_pro)
    cp_u.wait()
    cp_p.wait()
    cp_bu.wait()
    cp_bp.wait()

    lane = lax.iota(jnp.int32, 16)
    lane0 = lane == 0

    def issue(u, slot):
        sem = sems[slot]
        offp = pl.multiple_of((_sread(cup_v, u) - startp) & jnp.int32(-8), 8)
        offw = pl.multiple_of((_sread(cuw_v, u) - startw) & jnp.int32(-8), 8)
        pltpu.async_copy(ifac.at[idxp_v.at[pl.ds(offp, P_ROWS)]],
                         rows_p.at[slot], sem)
        pltpu.async_copy(iwl.at[idxw_v.at[pl.ds(offw, W_ROWS)]],
                         rows_w.at[slot], sem)

    def seg_sum(rows, slot, r0, n):
        zero = jnp.zeros((16,), jnp.float32)

        def ld(jr, c):
            return rows[slot, jr, pl.ds(c * 16, 16)]

        def bd4(q, acc):
            a = list(acc)
            jr = r0 + q * 4
            for t in range(4):
                for c in range(4):
                    a[c] = a[c] + ld(jr + t, c)
            return tuple(a)

        acc = lax.fori_loop(0, lax.shift_right_logical(n, 2), bd4,
                            (zero, zero, zero, zero))
        jb = r0 + (n & jnp.int32(-4))
        nt = n & jnp.int32(3)
        a = list(acc)
        for t in range(3):
            w = jnp.where(t < nt, 1.0, 0.0).astype(jnp.float32)
            for c in range(4):
                a[c] = a[c] + ld(jb + t, c) * w
        return tuple(a)

    def consume(u, slot):
        sem = sems[slot]
        pltpu.make_async_copy(ifac.at[pl.ds(0, P_ROWS)],
                              rows_p.at[slot], sem).wait()
        pltpu.make_async_copy(iwl.at[pl.ds(0, W_ROWS)],
                              rows_w.at[slot], sem).wait()
        sp = _sread(cup_v, u)
        lenp = _sread(cup_v, u + 1) - sp
        r0p = (sp - startp) & jnp.int32(7)
        sw = _sread(cuw_v, u)
        lenw = _sread(cuw_v, u + 1) - sw
        r0w = (sw - startw) & jnp.int32(7)
        accp = seg_sum(rows_p, slot, r0p, lenp)
        accw = seg_sum(rows_w, slot, r0w, lenw)
        rsp = _sread(lut_v, lenp)
        rsw = _sread(lut_v, lenw)
        tacc = jnp.zeros((16,), jnp.float32)
        for ci in range(4):
            sl = pl.ds(ci * 16, 16)
            y = accp[ci] * rsp + accw[ci] * rsw + u_v[u, sl]
            tacc = tacc + y * p_v[u, sl]
        dot = jnp.full((16,), jnp.sum(tacc))
        plsc.store_scatter(out_v, [jnp.full((16,), u, jnp.int32)], dot,
                           mask=lane0)

    for s in range(NSLOT - 1):
        issue(jnp.int32(s), s)

    def outer(g, carry):
        for par in range(NSLOT):
            u = g * NSLOT + par

            @pl.when(u + NSLOT - 1 < UPW)
            def _():
                issue(u + (NSLOT - 1), (par + NSLOT - 1) % NSLOT)

            consume(u, par)
        return carry

    lax.fori_loop(0, UPW // NSLOT, outer, 0)

    gbias = _sread(gb_v, 0)
    mask15 = jnp.full((16,), 15, jnp.int32)
    for k in range(UPW // 16):
        sl = pl.ds(k * 16, 16)
        rows16 = lane + (k * 16)
        bu16 = plsc.load_gather(bu_v, [rows16, sid_v[sl] & mask15])
        bp16 = plsc.load_gather(bp_v, [rows16, pid_v[sl] & mask15])
        out_v[sl] = out_v[sl] + bu16 + bp16 + gbias

    pltpu.sync_copy(out_v, out_hbm.at[pl.ds(base, UPW)])


_scall = functools.partial(
    pl.kernel,
    out_type=jax.ShapeDtypeStruct((B,), jnp.float32),
    mesh=plsc.VectorSubcoreMesh(core_axis_name="c", subcore_axis_name="s",
                                num_cores=NC, num_subcores=NS),
    compiler_params=pltpu.CompilerParams(needs_layout_passes=False, use_tc_tiling_on_sc=False),
    scratch_types=[
        pltpu.VMEM((IDXP,), jnp.int32),
        pltpu.VMEM((IDXW,), jnp.int32),
        pltpu.VMEM((CU_LEN,), jnp.int32),
        pltpu.VMEM((CU_LEN,), jnp.int32),
        pltpu.VMEM((UPW,), jnp.int32),
        pltpu.VMEM((UPW,), jnp.int32),
        pltpu.VMEM((UPW,), jnp.int32),
        pltpu.VMEM((UPW,), jnp.int32),
        pltpu.VMEM((UPW, EMB), jnp.float32),
        pltpu.VMEM((UPW, EMB), jnp.float32),
        pltpu.VMEM((UPW, 16), jnp.float32),
        pltpu.VMEM((UPW, 16), jnp.float32),
        pltpu.VMEM((LUT_LEN,), jnp.float32),
        pltpu.VMEM((16,), jnp.float32),
        pltpu.VMEM((UPW,), jnp.float32),
        pltpu.VMEM((NSLOT, P_ROWS, EMB), jnp.float32),
        pltpu.VMEM((NSLOT, W_ROWS, EMB), jnp.float32),
        pltpu.SemaphoreType.DMA,
        pltpu.SemaphoreType.DMA,
        pltpu.SemaphoreType.DMA,
        pltpu.SemaphoreType.DMA,
        pltpu.SemaphoreType.DMA,
    ],
)(_body)


def kernel(scientist_ids, paper_ids, scientist_factors, paper_factors,
           scientist_bias, paper_bias, implicit_factors, implicit_wishlist,
           global_bias, s2p_flat, s2p_cu, s2w_flat, s2w_cu):
    s2p_pad = jnp.pad(s2p_flat, (0, IDXP))
    s2w_pad = jnp.pad(s2w_flat, (0, IDXW))
    cup_pad = jnp.pad(s2p_cu, (0, CU_LEN))
    cuw_pad = jnp.pad(s2w_cu, (0, CU_LEN))
    gb_pad = jnp.pad(global_bias, (0, 15))
    lut = jnp.asarray(
        1.0 / np.sqrt(np.maximum(np.arange(LUT_LEN), 1)), jnp.float32)
    sb16 = scientist_bias.reshape(-1, 16)
    pb16 = paper_bias.reshape(-1, 16)
    return _scall(scientist_ids.astype(jnp.int32), paper_ids.astype(jnp.int32),
                  scientist_factors, paper_factors, sb16, pb16,
                  implicit_factors, implicit_wishlist, gb_pad, s2p_pad,
                  cup_pad, s2w_pad, cuw_pad, lut)
